# baseline (speedup 1.0000x reference)
import functools

import jax
import jax.numpy as jnp
from jax.experimental import pallas as pl
from jax.experimental.pallas import tpu as pltpu

_BATCH = 1024
_MEM = 200
_DIM = 128
_BB = 64


def _tc_body(q_ref, m_ref, soft_ref, logit_ref):
    q = q_ref[...]
    m = m_ref[...]
    logits = jnp.sum(m * q[:, None, :], axis=2)
    logit_ref[...] = logits
    mx = jnp.max(logits, axis=1, keepdims=True)
    e = jnp.exp(logits - mx)
    soft_ref[...] = e / jnp.sum(e, axis=1, keepdims=True)


@jax.jit
def kernel(query_vector, graph_out_features):
    grid = (_BATCH // _BB,)
    out_shape = [
        jax.ShapeDtypeStruct((_BATCH, _MEM), jnp.float32),
        jax.ShapeDtypeStruct((_BATCH, _MEM), jnp.float32),
    ]
    soft, logits = pl.pallas_call(
        _tc_body,
        grid=grid,
        in_specs=[
            pl.BlockSpec((_BB, _DIM), lambda i: (i, 0)),
            pl.BlockSpec((_BB, _MEM, _DIM), lambda i: (i, 0, 0)),
        ],
        out_specs=[
            pl.BlockSpec((_BB, _MEM), lambda i: (i, 0)),
            pl.BlockSpec((_BB, _MEM), lambda i: (i, 0)),
        ],
        out_shape=out_shape,
    )(query_vector, graph_out_features)
    return (soft, logits)


# speedup vs baseline: 1.8621x; 1.8621x over previous
import functools

import jax
import jax.numpy as jnp
from jax.experimental import pallas as pl
from jax.experimental.pallas import tpu as pltpu

_BATCH = 1024
_MEM = 200
_DIM = 128
_BB = 64


def _tc_body(q_ref, m_ref, soft_ref, logit_ref):
    q = q_ref[...]
    m = m_ref[...]
    x = m * q[:, None, :]
    xt = jnp.swapaxes(x, 1, 2)
    lt3 = jnp.sum(xt, axis=1)
    logit_ref[...] = lt3
    lt = lt3.T
    mx = jnp.max(lt, axis=0, keepdims=True)
    e = jnp.exp(lt - mx)
    st = e / jnp.sum(e, axis=0, keepdims=True)
    soft_ref[...] = st.T


@jax.jit
def kernel(query_vector, graph_out_features):
    grid = (_BATCH // _BB,)
    out_shape = [
        jax.ShapeDtypeStruct((_BATCH, _MEM), jnp.float32),
        jax.ShapeDtypeStruct((_BATCH, _MEM), jnp.float32),
    ]
    soft, logits = pl.pallas_call(
        _tc_body,
        grid=grid,
        in_specs=[
            pl.BlockSpec((_BB, _DIM), lambda i: (i, 0)),
            pl.BlockSpec((_BB, _MEM, _DIM), lambda i: (i, 0, 0)),
        ],
        out_specs=[
            pl.BlockSpec((_BB, _MEM), lambda i: (i, 0)),
            pl.BlockSpec((_BB, _MEM), lambda i: (i, 0)),
        ],
        out_shape=out_shape,
    )(query_vector, graph_out_features)
    return (soft, logits)


# speedup vs baseline: 2.0363x; 1.0936x over previous
import functools

import jax
import jax.numpy as jnp
from jax.experimental import pallas as pl
from jax.experimental.pallas import tpu as pltpu

_BATCH = 1024
_MEM = 200
_DIM = 128
_BB = 64


def _tc_body(q_ref, m_ref, soft_ref, logit_ref, lt_ref):
    qt = q_ref[...].T
    for b in range(_BB):
        lt_ref[:, b : b + 1] = jax.lax.dot(
            m_ref[b], qt[:, b : b + 1], preferred_element_type=jnp.float32
        )
    lt = lt_ref[...]
    logit_ref[...] = lt.T
    mx = jnp.max(lt, axis=0, keepdims=True)
    e = jnp.exp(lt - mx)
    st = e / jnp.sum(e, axis=0, keepdims=True)
    soft_ref[...] = st.T


@jax.jit
def kernel(query_vector, graph_out_features):
    grid = (_BATCH // _BB,)
    out_shape = [
        jax.ShapeDtypeStruct((_BATCH, _MEM), jnp.float32),
        jax.ShapeDtypeStruct((_BATCH, _MEM), jnp.float32),
    ]
    soft, logits = pl.pallas_call(
        _tc_body,
        grid=grid,
        in_specs=[
            pl.BlockSpec((_BB, _DIM), lambda i: (i, 0)),
            pl.BlockSpec((_BB, _MEM, _DIM), lambda i: (i, 0, 0)),
        ],
        out_specs=[
            pl.BlockSpec((_BB, _MEM), lambda i: (i, 0)),
            pl.BlockSpec((_BB, _MEM), lambda i: (i, 0)),
        ],
        out_shape=out_shape,
        scratch_shapes=[pltpu.VMEM((_MEM, _BB), jnp.float32)],
    )(query_vector, graph_out_features)
    return (soft, logits)
